# initial kernel scaffold (unmeasured)
import jax
import jax.numpy as jnp
from jax import lax
from jax.experimental import pallas as pl
from jax.experimental.pallas import tpu as pltpu

T = 2048
D = 1024
F = 2048
E_LOCAL = 4
CHUNK = 1024
N_CHUNKS = T // CHUNK


def kernel(x, assign, W1, W2):
    xb = x.astype(jnp.bfloat16)
    w1b = W1.astype(jnp.bfloat16)
    w2b = W2.astype(jnp.bfloat16)
    a2d = assign.reshape(T, 1)

    def body(x_ref, a_ref, w1_ref, w2_ref, out_ref,
             xpeer, apeer, sendbuf, recvbuf, send_sems, recv_sems):
        my_x = lax.axis_index("x")
        my_y = lax.axis_index("y")
        peer = (my_x, 1 - my_y)

        barrier_sem = pltpu.get_barrier_semaphore()
        pl.semaphore_signal(barrier_sem, inc=1, device_id=peer,
                            device_id_type=pl.DeviceIdType.MESH)
        pl.semaphore_wait(barrier_sem, 1)

        rdma_x = pltpu.make_async_remote_copy(
            src_ref=x_ref, dst_ref=xpeer,
            send_sem=send_sems.at[0], recv_sem=recv_sems.at[0],
            device_id=peer, device_id_type=pl.DeviceIdType.MESH)
        rdma_a = pltpu.make_async_remote_copy(
            src_ref=a_ref, dst_ref=apeer,
            send_sem=send_sems.at[1], recv_sem=recv_sems.at[1],
            device_id=peer, device_id_type=pl.DeviceIdType.MESH)
        rdma_x.start()
        rdma_a.start()

        def ffn_chunk(xc, ac):
            tot = None
            for e in range(E_LOCAL):
                ge = my_y * E_LOCAL + e
                xm = jnp.where(ac == ge, xc, jnp.zeros_like(xc))
                h = jnp.dot(xm, w1_ref[e], preferred_element_type=jnp.float32)
                hb = jnp.maximum(h, 0.0).astype(jnp.bfloat16)
                d2 = jnp.dot(hb, w2_ref[e], preferred_element_type=jnp.float32)
                tot = d2 if tot is None else tot + d2
            return tot

        for c in range(N_CHUNKS):
            rows = pl.ds(c * CHUNK, CHUNK)
            out_ref[rows, :] = ffn_chunk(x_ref[rows, :], a_ref[rows, :])

        rdma_x.wait()
        rdma_a.wait()

        for c in range(N_CHUNKS):
            rows = pl.ds(c * CHUNK, CHUNK)
            sendbuf[rows, :] = ffn_chunk(
                xpeer[rows, :], apeer[rows, :]).astype(jnp.bfloat16)

        rdma_p = pltpu.make_async_remote_copy(
            src_ref=sendbuf, dst_ref=recvbuf,
            send_sem=send_sems.at[2], recv_sem=recv_sems.at[2],
            device_id=peer, device_id_type=pl.DeviceIdType.MESH)
        rdma_p.start()
        rdma_p.wait()

        out_ref[...] = out_ref[...] + recvbuf[...].astype(jnp.float32)

    return pl.pallas_call(
        body,
        out_shape=jax.ShapeDtypeStruct((T, D), jnp.float32),
        in_specs=[pl.BlockSpec(memory_space=pltpu.VMEM)] * 4,
        out_specs=pl.BlockSpec(memory_space=pltpu.VMEM),
        scratch_shapes=[
            pltpu.VMEM((T, D), jnp.bfloat16),
            pltpu.VMEM((T, 1), jnp.int32),
            pltpu.VMEM((T, D), jnp.bfloat16),
            pltpu.VMEM((T, D), jnp.bfloat16),
            pltpu.SemaphoreType.DMA((3,)),
            pltpu.SemaphoreType.DMA((3,)),
        ],
        compiler_params=pltpu.CompilerParams(collective_id=0),
    )(xb, a2d, w1b, w2b)


# baseline (device time: 266671 ns/iter reference)
import jax
import jax.numpy as jnp
from jax import lax
from jax.experimental import pallas as pl
from jax.experimental.pallas import tpu as pltpu

T = 2048
D = 1024
F = 2048
E_LOCAL = 4
CHUNK = 512
N_CHUNKS = T // CHUNK


def kernel(x, assign, W1, W2):
    xb = x.astype(jnp.bfloat16)
    w1b = W1.astype(jnp.bfloat16)
    w2b = W2.astype(jnp.bfloat16)
    a2d = assign.reshape(T, 1)

    def body(x_ref, a_ref, w1_hbm, w2_hbm, out_ref,
             xpeer, apeer, sendbuf, recvbuf,
             w1buf, w2buf, send_sems, recv_sems, w1sems, w2sems):
        my_x = lax.axis_index("x")
        my_y = lax.axis_index("y")
        peer = (my_x, 1 - my_y)

        barrier_sem = pltpu.get_barrier_semaphore()
        pl.semaphore_signal(barrier_sem, inc=1, device_id=peer,
                            device_id_type=pl.DeviceIdType.MESH)
        pl.semaphore_wait(barrier_sem, 1)

        rdma_x = pltpu.make_async_remote_copy(
            src_ref=x_ref, dst_ref=xpeer,
            send_sem=send_sems.at[0], recv_sem=recv_sems.at[0],
            device_id=peer, device_id_type=pl.DeviceIdType.MESH)
        rdma_a = pltpu.make_async_remote_copy(
            src_ref=a_ref, dst_ref=apeer,
            send_sem=send_sems.at[1], recv_sem=recv_sems.at[1],
            device_id=peer, device_id_type=pl.DeviceIdType.MESH)
        rdma_x.start()
        rdma_a.start()

        def load_weights(e, slot):
            c1 = pltpu.make_async_copy(
                w1_hbm.at[e], w1buf.at[slot], w1sems.at[slot])
            c2 = pltpu.make_async_copy(
                w2_hbm.at[e], w2buf.at[slot], w2sems.at[slot])
            c1.start()
            c2.start()
            return c1, c2

        def expert_pass(e, slot, src_x, src_a, dst, dst_dtype):
            for c in range(N_CHUNKS):
                rows = pl.ds(c * CHUNK, CHUNK)
                xc = src_x[rows, :]
                ac = src_a[rows, :]
                ge = my_y * E_LOCAL + e
                xm = jnp.where(ac == ge, xc, jnp.zeros_like(xc))
                h = jnp.dot(xm, w1buf[slot],
                            preferred_element_type=jnp.float32)
                hb = jnp.maximum(h, 0.0).astype(jnp.bfloat16)
                d2 = jnp.dot(hb, w2buf[slot],
                             preferred_element_type=jnp.float32)
                contrib = d2.astype(dst_dtype)
                if e == 0:
                    dst[rows, :] = contrib
                else:
                    dst[rows, :] = dst[rows, :] + contrib

        for phase in range(2):
            if phase == 1:
                rdma_x.wait()
                rdma_a.wait()
            src_x = x_ref if phase == 0 else xpeer
            src_a = a_ref if phase == 0 else apeer
            dst = out_ref if phase == 0 else sendbuf
            dst_dtype = jnp.float32 if phase == 0 else jnp.bfloat16
            for e in range(E_LOCAL):
                slot = e % 2
                c1, c2 = load_weights(e, slot)
                c1.wait()
                c2.wait()
                expert_pass(e, slot, src_x, src_a, dst, dst_dtype)

        rdma_p = pltpu.make_async_remote_copy(
            src_ref=sendbuf, dst_ref=recvbuf,
            send_sem=send_sems.at[2], recv_sem=recv_sems.at[2],
            device_id=peer, device_id_type=pl.DeviceIdType.MESH)
        rdma_p.start()
        rdma_p.wait()

        out_ref[...] = out_ref[...] + recvbuf[...].astype(jnp.float32)

    return pl.pallas_call(
        body,
        out_shape=jax.ShapeDtypeStruct((T, D), jnp.float32),
        in_specs=[
            pl.BlockSpec(memory_space=pltpu.VMEM),
            pl.BlockSpec(memory_space=pltpu.VMEM),
            pl.BlockSpec(memory_space=pl.ANY),
            pl.BlockSpec(memory_space=pl.ANY),
        ],
        out_specs=pl.BlockSpec(memory_space=pltpu.VMEM),
        scratch_shapes=[
            pltpu.VMEM((T, D), jnp.bfloat16),
            pltpu.VMEM((T, 1), jnp.int32),
            pltpu.VMEM((T, D), jnp.bfloat16),
            pltpu.VMEM((T, D), jnp.bfloat16),
            pltpu.VMEM((2, D, F), jnp.bfloat16),
            pltpu.VMEM((2, F, D), jnp.bfloat16),
            pltpu.SemaphoreType.DMA((3,)),
            pltpu.SemaphoreType.DMA((3,)),
            pltpu.SemaphoreType.DMA((2,)),
            pltpu.SemaphoreType.DMA((2,)),
        ],
        compiler_params=pltpu.CompilerParams(
            collective_id=0, vmem_limit_bytes=48 * 1024 * 1024),
    )(xb, a2d, w1b, w2b)


# device time: 216292 ns/iter; 1.2329x vs baseline; 1.2329x over previous
import jax
import jax.numpy as jnp
from jax import lax
from jax.experimental import pallas as pl
from jax.experimental.pallas import tpu as pltpu

T = 2048
D = 1024
F = 2048
E_LOCAL = 4
CHUNK = 512
N_CHUNKS = T // CHUNK


def kernel(x, assign, W1, W2):
    xb = x.astype(jnp.bfloat16)
    w1b = W1.astype(jnp.bfloat16)
    w2b = W2.astype(jnp.bfloat16)
    a2d = assign.reshape(T, 1)

    def body(x_ref, a_ref, w1_hbm, w2_hbm, out_ref,
             xpeer, apeer, sendbuf, recvbuf,
             w1buf, w2buf, send_sems, recv_sems, w1sems, w2sems):
        my_x = lax.axis_index("x")
        my_y = lax.axis_index("y")
        peer = (my_x, 1 - my_y)

        barrier_sem = pltpu.get_barrier_semaphore()
        pl.semaphore_signal(barrier_sem, inc=1, device_id=peer,
                            device_id_type=pl.DeviceIdType.MESH)
        pl.semaphore_wait(barrier_sem, 1)

        rdma_x = pltpu.make_async_remote_copy(
            src_ref=x_ref, dst_ref=xpeer,
            send_sem=send_sems.at[0], recv_sem=recv_sems.at[0],
            device_id=peer, device_id_type=pl.DeviceIdType.MESH)
        rdma_a = pltpu.make_async_remote_copy(
            src_ref=a_ref, dst_ref=apeer,
            send_sem=send_sems.at[1], recv_sem=recv_sems.at[1],
            device_id=peer, device_id_type=pl.DeviceIdType.MESH)
        rdma_x.start()
        rdma_a.start()

        loads = list(range(E_LOCAL)) + [
            e for _ in range(N_CHUNKS) for e in range(E_LOCAL)]

        def load_copies(i):
            slot = i % 2
            return (
                pltpu.make_async_copy(
                    w1_hbm.at[loads[i]], w1buf.at[slot], w1sems.at[slot]),
                pltpu.make_async_copy(
                    w2_hbm.at[loads[i]], w2buf.at[slot], w2sems.at[slot]),
            )

        def start_load(i):
            c1, c2 = load_copies(i)
            c1.start()
            c2.start()

        def wait_load(i):
            c1, c2 = load_copies(i)
            c1.wait()
            c2.wait()

        def expert_chunk(e, slot, src_x, src_a, rows):
            xc = src_x[rows, :]
            ac = src_a[rows, :]
            ge = my_y * E_LOCAL + e
            xm = jnp.where(ac == ge, xc, jnp.zeros_like(xc))
            h = jnp.dot(xm, w1buf[slot], preferred_element_type=jnp.float32)
            hb = jnp.maximum(h, 0.0).astype(jnp.bfloat16)
            return jnp.dot(hb, w2buf[slot],
                           preferred_element_type=jnp.float32)

        n_loads = len(loads)
        start_load(0)
        step = 0

        for e in range(E_LOCAL):
            if step + 1 < n_loads:
                start_load(step + 1)
            wait_load(step)
            slot = step % 2
            for c in range(N_CHUNKS):
                rows = pl.ds(c * CHUNK, CHUNK)
                d2 = expert_chunk(e, slot, x_ref, a_ref, rows)
                if e == 0:
                    out_ref[rows, :] = d2
                else:
                    out_ref[rows, :] = out_ref[rows, :] + d2
            step += 1

        rdma_x.wait()
        rdma_a.wait()

        partial_rdmas = []
        for c in range(N_CHUNKS):
            rows = pl.ds(c * CHUNK, CHUNK)
            for e in range(E_LOCAL):
                if step + 1 < n_loads:
                    start_load(step + 1)
                wait_load(step)
                slot = step % 2
                d2 = expert_chunk(e, slot, xpeer, apeer, rows)
                if e == 0:
                    sendbuf[c, :, :] = d2.astype(jnp.bfloat16)
                else:
                    sendbuf[c, :, :] = sendbuf[c, :, :] + d2.astype(
                        jnp.bfloat16)
                step += 1
            rdma_p = pltpu.make_async_remote_copy(
                src_ref=sendbuf.at[c], dst_ref=recvbuf.at[c],
                send_sem=send_sems.at[2 + c], recv_sem=recv_sems.at[2 + c],
                device_id=peer, device_id_type=pl.DeviceIdType.MESH)
            rdma_p.start()
            partial_rdmas.append(rdma_p)

        for c in range(N_CHUNKS):
            partial_rdmas[c].wait()
            rows = pl.ds(c * CHUNK, CHUNK)
            out_ref[rows, :] = out_ref[rows, :] + recvbuf[c].astype(
                jnp.float32)

    return pl.pallas_call(
        body,
        out_shape=jax.ShapeDtypeStruct((T, D), jnp.float32),
        in_specs=[
            pl.BlockSpec(memory_space=pltpu.VMEM),
            pl.BlockSpec(memory_space=pltpu.VMEM),
            pl.BlockSpec(memory_space=pl.ANY),
            pl.BlockSpec(memory_space=pl.ANY),
        ],
        out_specs=pl.BlockSpec(memory_space=pltpu.VMEM),
        scratch_shapes=[
            pltpu.VMEM((T, D), jnp.bfloat16),
            pltpu.VMEM((T, 1), jnp.int32),
            pltpu.VMEM((N_CHUNKS, CHUNK, D), jnp.bfloat16),
            pltpu.VMEM((N_CHUNKS, CHUNK, D), jnp.bfloat16),
            pltpu.VMEM((2, D, F), jnp.bfloat16),
            pltpu.VMEM((2, F, D), jnp.bfloat16),
            pltpu.SemaphoreType.DMA((2 + N_CHUNKS,)),
            pltpu.SemaphoreType.DMA((2 + N_CHUNKS,)),
            pltpu.SemaphoreType.DMA((2,)),
            pltpu.SemaphoreType.DMA((2,)),
        ],
        compiler_params=pltpu.CompilerParams(
            collective_id=0, vmem_limit_bytes=48 * 1024 * 1024),
    )(xb, a2d, w1b, w2b)


# device time: 157487 ns/iter; 1.6933x vs baseline; 1.3734x over previous
import jax
import jax.numpy as jnp
from jax import lax
from jax.experimental import pallas as pl
from jax.experimental.pallas import tpu as pltpu

T = 2048
D = 1024
F = 2048
E_LOCAL = 4
HALF = T // 2
CHUNK = 512
N_CHUNKS = HALF // CHUNK


def kernel(x, assign, W1, W2):
    xb = x.astype(jnp.bfloat16)
    w1b = W1.astype(jnp.bfloat16)
    w2b = W2.astype(jnp.bfloat16)
    a2d = assign.reshape(T, 1)

    def body(x_ref, a_ref, w1_hbm, w2_hbm, out_ref,
             xstage, astage, xpeer, apeer, sendbuf, recvbuf, xsend, xrecv,
             w1buf, w2buf, send_sems, recv_sems, w1sems, w2sems):
        my_x = lax.axis_index("x")
        my_y = lax.axis_index("y")
        peer_y = (my_x, 1 - my_y)
        peer_x = (1 - my_x, my_y)
        base = my_x * HALF
        obase = (1 - my_x) * HALF

        barrier_sem = pltpu.get_barrier_semaphore()
        for nbr in (peer_y, peer_x):
            pl.semaphore_signal(barrier_sem, inc=1, device_id=nbr,
                                device_id_type=pl.DeviceIdType.MESH)
        pl.semaphore_wait(barrier_sem, 2)

        xstage[...] = x_ref[pl.ds(base, HALF), :]
        astage[...] = a_ref[pl.ds(base, HALF), :]
        rdma_x = pltpu.make_async_remote_copy(
            src_ref=xstage, dst_ref=xpeer,
            send_sem=send_sems.at[0], recv_sem=recv_sems.at[0],
            device_id=peer_y, device_id_type=pl.DeviceIdType.MESH)
        rdma_a = pltpu.make_async_remote_copy(
            src_ref=astage, dst_ref=apeer,
            send_sem=send_sems.at[1], recv_sem=recv_sems.at[1],
            device_id=peer_y, device_id_type=pl.DeviceIdType.MESH)
        rdma_x.start()
        rdma_a.start()

        loads = list(range(E_LOCAL)) + [
            e for _ in range(N_CHUNKS) for e in range(E_LOCAL)]

        def load_copies(i):
            slot = i % 2
            return (
                pltpu.make_async_copy(
                    w1_hbm.at[loads[i]], w1buf.at[slot], w1sems.at[slot]),
                pltpu.make_async_copy(
                    w2_hbm.at[loads[i]], w2buf.at[slot], w2sems.at[slot]),
            )

        def start_load(i):
            c1, c2 = load_copies(i)
            c1.start()
            c2.start()

        def wait_load(i):
            c1, c2 = load_copies(i)
            c1.wait()
            c2.wait()

        def expert_chunk(e, slot, xc, ac):
            ge = my_y * E_LOCAL + e
            xm = jnp.where(ac == ge, xc, jnp.zeros_like(xc))
            h = jnp.dot(xm, w1buf[slot], preferred_element_type=jnp.float32)
            hb = jnp.maximum(h, 0.0).astype(jnp.bfloat16)
            return jnp.dot(hb, w2buf[slot],
                           preferred_element_type=jnp.float32)

        n_loads = len(loads)
        start_load(0)
        step = 0

        for e in range(E_LOCAL):
            if step + 1 < n_loads:
                start_load(step + 1)
            wait_load(step)
            slot = step % 2
            for c in range(N_CHUNKS):
                rows = pl.ds(base + c * CHUNK, CHUNK)
                d2 = expert_chunk(e, slot, x_ref[rows, :], a_ref[rows, :])
                if e == 0:
                    out_ref[rows, :] = d2
                else:
                    out_ref[rows, :] = out_ref[rows, :] + d2
            step += 1

        rdma_x.wait()
        rdma_a.wait()

        partial_rdmas = []
        for c in range(N_CHUNKS):
            rows = pl.ds(c * CHUNK, CHUNK)
            for e in range(E_LOCAL):
                if step + 1 < n_loads:
                    start_load(step + 1)
                wait_load(step)
                slot = step % 2
                d2 = expert_chunk(e, slot, xpeer[rows, :], apeer[rows, :])
                if e == 0:
                    sendbuf[c, :, :] = d2.astype(jnp.bfloat16)
                else:
                    sendbuf[c, :, :] = sendbuf[c, :, :] + d2.astype(
                        jnp.bfloat16)
                step += 1
            rdma_p = pltpu.make_async_remote_copy(
                src_ref=sendbuf.at[c], dst_ref=recvbuf.at[c],
                send_sem=send_sems.at[2 + c], recv_sem=recv_sems.at[2 + c],
                device_id=peer_y, device_id_type=pl.DeviceIdType.MESH)
            rdma_p.start()
            partial_rdmas.append(rdma_p)

        xtotal_rdmas = []
        for c in range(N_CHUNKS):
            partial_rdmas[c].wait()
            rows = pl.ds(base + c * CHUNK, CHUNK)
            tot = out_ref[rows, :] + recvbuf[c].astype(jnp.float32)
            out_ref[rows, :] = tot
            xsend[c, :, :] = tot.astype(jnp.bfloat16)
            rdma_t = pltpu.make_async_remote_copy(
                src_ref=xsend.at[c], dst_ref=xrecv.at[c],
                send_sem=send_sems.at[4 + c], recv_sem=recv_sems.at[4 + c],
                device_id=peer_x, device_id_type=pl.DeviceIdType.MESH)
            rdma_t.start()
            xtotal_rdmas.append(rdma_t)

        for c in range(N_CHUNKS):
            xtotal_rdmas[c].wait()
            rows = pl.ds(obase + c * CHUNK, CHUNK)
            out_ref[rows, :] = xrecv[c].astype(jnp.float32)

    return pl.pallas_call(
        body,
        out_shape=jax.ShapeDtypeStruct((T, D), jnp.float32),
        in_specs=[
            pl.BlockSpec(memory_space=pltpu.VMEM),
            pl.BlockSpec(memory_space=pltpu.VMEM),
            pl.BlockSpec(memory_space=pl.ANY),
            pl.BlockSpec(memory_space=pl.ANY),
        ],
        out_specs=pl.BlockSpec(memory_space=pltpu.VMEM),
        scratch_shapes=[
            pltpu.VMEM((HALF, D), jnp.bfloat16),
            pltpu.VMEM((HALF, 1), jnp.int32),
            pltpu.VMEM((HALF, D), jnp.bfloat16),
            pltpu.VMEM((HALF, 1), jnp.int32),
            pltpu.VMEM((N_CHUNKS, CHUNK, D), jnp.bfloat16),
            pltpu.VMEM((N_CHUNKS, CHUNK, D), jnp.bfloat16),
            pltpu.VMEM((N_CHUNKS, CHUNK, D), jnp.bfloat16),
            pltpu.VMEM((N_CHUNKS, CHUNK, D), jnp.bfloat16),
            pltpu.VMEM((2, D, F), jnp.bfloat16),
            pltpu.VMEM((2, F, D), jnp.bfloat16),
            pltpu.SemaphoreType.DMA((4 + N_CHUNKS,)),
            pltpu.SemaphoreType.DMA((4 + N_CHUNKS,)),
            pltpu.SemaphoreType.DMA((2,)),
            pltpu.SemaphoreType.DMA((2,)),
        ],
        compiler_params=pltpu.CompilerParams(
            collective_id=0, vmem_limit_bytes=48 * 1024 * 1024),
    )(xb, a2d, w1b, w2b)


# device time: 155555 ns/iter; 1.7143x vs baseline; 1.0124x over previous
import jax
import jax.numpy as jnp
from jax import lax
from jax.experimental import pallas as pl
from jax.experimental.pallas import tpu as pltpu

T = 2048
D = 1024
F = 2048
E_LOCAL = 4
HALF = T // 2
CHUNK = 512
N_CHUNKS = HALF // CHUNK
PCHUNK = 256
NP = HALF // PCHUNK


def kernel(x, assign, W1, W2):
    xb = x.astype(jnp.bfloat16)
    w1b = W1.astype(jnp.bfloat16)
    w2b = W2.astype(jnp.bfloat16)
    a2d = assign.reshape(T, 1)

    def body(x_ref, a_ref, w1_hbm, w2_hbm, out_ref,
             xstage, astage, xpeer, apeer, sendbuf, recvbuf, xsend, xrecv,
             w1buf, w2buf, send_sems, recv_sems, w1sems, w2sems):
        my_x = lax.axis_index("x")
        my_y = lax.axis_index("y")
        peer_y = (my_x, 1 - my_y)
        peer_x = (1 - my_x, my_y)
        base = my_x * HALF
        obase = (1 - my_x) * HALF

        barrier_sem = pltpu.get_barrier_semaphore()
        for nbr in (peer_y, peer_x):
            pl.semaphore_signal(barrier_sem, inc=1, device_id=nbr,
                                device_id_type=pl.DeviceIdType.MESH)
        pl.semaphore_wait(barrier_sem, 2)

        xstage[...] = x_ref[pl.ds(base, HALF), :]
        astage[...] = a_ref[pl.ds(base, HALF), :]
        rdma_x = pltpu.make_async_remote_copy(
            src_ref=xstage, dst_ref=xpeer,
            send_sem=send_sems.at[0], recv_sem=recv_sems.at[0],
            device_id=peer_y, device_id_type=pl.DeviceIdType.MESH)
        rdma_a = pltpu.make_async_remote_copy(
            src_ref=astage, dst_ref=apeer,
            send_sem=send_sems.at[1], recv_sem=recv_sems.at[1],
            device_id=peer_y, device_id_type=pl.DeviceIdType.MESH)
        rdma_x.start()
        rdma_a.start()

        loads = list(range(E_LOCAL)) * 2

        def load_copies(i):
            slot = i % 2
            return (
                pltpu.make_async_copy(
                    w1_hbm.at[loads[i]], w1buf.at[slot], w1sems.at[slot]),
                pltpu.make_async_copy(
                    w2_hbm.at[loads[i]], w2buf.at[slot], w2sems.at[slot]),
            )

        def start_load(i):
            c1, c2 = load_copies(i)
            c1.start()
            c2.start()

        def wait_load(i):
            c1, c2 = load_copies(i)
            c1.wait()
            c2.wait()

        def expert_chunk(e, slot, xc, ac):
            ge = my_y * E_LOCAL + e
            xm = jnp.where(ac == ge, xc, jnp.zeros_like(xc))
            h = jnp.dot(xm, w1buf[slot], preferred_element_type=jnp.float32)
            hb = jnp.maximum(h, 0.0).astype(jnp.bfloat16)
            return jnp.dot(hb, w2buf[slot],
                           preferred_element_type=jnp.float32)

        n_loads = len(loads)
        start_load(0)
        step = 0

        for e in range(E_LOCAL):
            if step + 1 < n_loads:
                start_load(step + 1)
            wait_load(step)
            slot = step % 2
            for c in range(N_CHUNKS):
                rows = pl.ds(base + c * CHUNK, CHUNK)
                d2 = expert_chunk(e, slot,
                                  xstage[pl.ds(c * CHUNK, CHUNK), :],
                                  a_ref[rows, :])
                if e == 0:
                    out_ref[rows, :] = d2
                else:
                    out_ref[rows, :] = out_ref[rows, :] + d2
            step += 1

        rdma_x.wait()
        rdma_a.wait()

        partial_rdmas = []
        for e in range(E_LOCAL):
            if step + 1 < n_loads:
                start_load(step + 1)
            wait_load(step)
            slot = step % 2
            for c in range(NP):
                rows = pl.ds(c * PCHUNK, PCHUNK)
                d2 = expert_chunk(e, slot, xpeer[rows, :], apeer[rows, :])
                if e == 0:
                    sendbuf[c, :, :] = d2.astype(jnp.bfloat16)
                else:
                    sendbuf[c, :, :] = sendbuf[c, :, :] + d2.astype(
                        jnp.bfloat16)
                if e == E_LOCAL - 1:
                    rdma_p = pltpu.make_async_remote_copy(
                        src_ref=sendbuf.at[c], dst_ref=recvbuf.at[c],
                        send_sem=send_sems.at[2 + c],
                        recv_sem=recv_sems.at[2 + c],
                        device_id=peer_y,
                        device_id_type=pl.DeviceIdType.MESH)
                    rdma_p.start()
                    partial_rdmas.append(rdma_p)
            step += 1

        xtotal_rdmas = []
        for c in range(NP):
            partial_rdmas[c].wait()
            rows = pl.ds(base + c * PCHUNK, PCHUNK)
            tot = out_ref[rows, :] + recvbuf[c].astype(jnp.float32)
            out_ref[rows, :] = tot
            xsend[c, :, :] = tot.astype(jnp.bfloat16)
            rdma_t = pltpu.make_async_remote_copy(
                src_ref=xsend.at[c], dst_ref=xrecv.at[c],
                send_sem=send_sems.at[2 + NP + c],
                recv_sem=recv_sems.at[2 + NP + c],
                device_id=peer_x, device_id_type=pl.DeviceIdType.MESH)
            rdma_t.start()
            xtotal_rdmas.append(rdma_t)

        for c in range(NP):
            xtotal_rdmas[c].wait()
            rows = pl.ds(obase + c * PCHUNK, PCHUNK)
            out_ref[rows, :] = xrecv[c].astype(jnp.float32)

    return pl.pallas_call(
        body,
        out_shape=jax.ShapeDtypeStruct((T, D), jnp.float32),
        in_specs=[
            pl.BlockSpec(memory_space=pltpu.VMEM),
            pl.BlockSpec(memory_space=pltpu.VMEM),
            pl.BlockSpec(memory_space=pl.ANY),
            pl.BlockSpec(memory_space=pl.ANY),
        ],
        out_specs=pl.BlockSpec(memory_space=pltpu.VMEM),
        scratch_shapes=[
            pltpu.VMEM((HALF, D), jnp.bfloat16),
            pltpu.VMEM((HALF, 1), jnp.int32),
            pltpu.VMEM((HALF, D), jnp.bfloat16),
            pltpu.VMEM((HALF, 1), jnp.int32),
            pltpu.VMEM((NP, PCHUNK, D), jnp.bfloat16),
            pltpu.VMEM((NP, PCHUNK, D), jnp.bfloat16),
            pltpu.VMEM((NP, PCHUNK, D), jnp.bfloat16),
            pltpu.VMEM((NP, PCHUNK, D), jnp.bfloat16),
            pltpu.VMEM((2, D, F), jnp.bfloat16),
            pltpu.VMEM((2, F, D), jnp.bfloat16),
            pltpu.SemaphoreType.DMA((2 + 2 * NP,)),
            pltpu.SemaphoreType.DMA((2 + 2 * NP,)),
            pltpu.SemaphoreType.DMA((2,)),
            pltpu.SemaphoreType.DMA((2,)),
        ],
        compiler_params=pltpu.CompilerParams(
            collective_id=0, vmem_limit_bytes=48 * 1024 * 1024),
    )(xb, a2d, w1b, w2b)


# device time: 126211 ns/iter; 2.1129x vs baseline; 1.2325x over previous
import jax
import jax.numpy as jnp
from jax import lax
from jax.experimental import pallas as pl
from jax.experimental.pallas import tpu as pltpu

T = 2048
D = 1024
F = 2048
FH = F // 2
E_LOCAL = 4
HALF = T // 2
CHUNK = 512
N_CHUNKS = HALF // CHUNK
PCHUNK = 256
NP = HALF // PCHUNK


def kernel(x, assign, W1, W2):
    a2d = assign.reshape(T, 1)

    def body(x_ref, a_ref, w1_hbm, w2_hbm, out_ref,
             xstage, astage, xpeer, apeer, sendbuf, recvbuf, xsend, xrecv,
             w1buf, w2buf, send_sems, recv_sems, w1sems, w2sems):
        my_x = lax.axis_index("x")
        my_y = lax.axis_index("y")
        peer_y = (my_x, 1 - my_y)
        peer_x = (1 - my_x, my_y)
        base = my_x * HALF
        obase = (1 - my_x) * HALF

        barrier_sem = pltpu.get_barrier_semaphore()
        for nbr in (peer_y, peer_x):
            pl.semaphore_signal(barrier_sem, inc=1, device_id=nbr,
                                device_id_type=pl.DeviceIdType.MESH)
        pl.semaphore_wait(barrier_sem, 2)

        xstage[...] = x_ref[pl.ds(base, HALF), :].astype(jnp.bfloat16)
        astage[...] = a_ref[pl.ds(base, HALF), :]
        rdma_x = pltpu.make_async_remote_copy(
            src_ref=xstage, dst_ref=xpeer,
            send_sem=send_sems.at[0], recv_sem=recv_sems.at[0],
            device_id=peer_y, device_id_type=pl.DeviceIdType.MESH)
        rdma_a = pltpu.make_async_remote_copy(
            src_ref=astage, dst_ref=apeer,
            send_sem=send_sems.at[1], recv_sem=recv_sems.at[1],
            device_id=peer_y, device_id_type=pl.DeviceIdType.MESH)
        rdma_x.start()
        rdma_a.start()

        loads = [(e, fh) for e in range(E_LOCAL) for fh in range(2)] * 2

        def load_copies(i):
            slot = i % 2
            e, fh = loads[i]
            return (
                pltpu.make_async_copy(
                    w1_hbm.at[e, :, pl.ds(fh * FH, FH)],
                    w1buf.at[slot], w1sems.at[slot]),
                pltpu.make_async_copy(
                    w2_hbm.at[e, pl.ds(fh * FH, FH), :],
                    w2buf.at[slot], w2sems.at[slot]),
            )

        def start_load(i):
            c1, c2 = load_copies(i)
            c1.start()
            c2.start()

        def wait_load(i):
            c1, c2 = load_copies(i)
            c1.wait()
            c2.wait()

        def expert_chunk(e, slot, xc, ac):
            ge = my_y * E_LOCAL + e
            xm = jnp.where(ac == ge, xc, jnp.zeros_like(xc))
            h = jnp.dot(xm, w1buf[slot], preferred_element_type=jnp.float32)
            hb = jnp.maximum(h, 0.0)
            return jnp.dot(hb, w2buf[slot],
                           preferred_element_type=jnp.float32)

        n_loads = len(loads)
        start_load(0)
        step = 0

        for e in range(E_LOCAL):
            for fh in range(2):
                if step + 1 < n_loads:
                    start_load(step + 1)
                wait_load(step)
                slot = step % 2
                for c in range(N_CHUNKS):
                    rows = pl.ds(base + c * CHUNK, CHUNK)
                    d2 = expert_chunk(e, slot, x_ref[rows, :],
                                      a_ref[rows, :])
                    if step == 0:
                        out_ref[rows, :] = d2
                    else:
                        out_ref[rows, :] = out_ref[rows, :] + d2
                step += 1

        rdma_x.wait()
        rdma_a.wait()

        partial_rdmas = []
        for e in range(E_LOCAL):
            for fh in range(2):
                if step + 1 < n_loads:
                    start_load(step + 1)
                wait_load(step)
                slot = step % 2
                for c in range(NP):
                    rows = pl.ds(c * PCHUNK, PCHUNK)
                    d2 = expert_chunk(
                        e, slot,
                        xpeer[rows, :].astype(jnp.float32),
                        apeer[rows, :])
                    if step == 8:
                        sendbuf[c, :, :] = d2.astype(jnp.bfloat16)
                    else:
                        sendbuf[c, :, :] = sendbuf[c, :, :] + d2.astype(
                            jnp.bfloat16)
                    if step == n_loads - 1:
                        rdma_p = pltpu.make_async_remote_copy(
                            src_ref=sendbuf.at[c], dst_ref=recvbuf.at[c],
                            send_sem=send_sems.at[2 + c],
                            recv_sem=recv_sems.at[2 + c],
                            device_id=peer_y,
                            device_id_type=pl.DeviceIdType.MESH)
                        rdma_p.start()
                        partial_rdmas.append(rdma_p)
                step += 1

        xtotal_rdmas = []
        for c in range(NP):
            partial_rdmas[c].wait()
            rows = pl.ds(base + c * PCHUNK, PCHUNK)
            tot = out_ref[rows, :] + recvbuf[c].astype(jnp.float32)
            out_ref[rows, :] = tot
            xsend[c, :, :] = tot.astype(jnp.bfloat16)
            rdma_t = pltpu.make_async_remote_copy(
                src_ref=xsend.at[c], dst_ref=xrecv.at[c],
                send_sem=send_sems.at[2 + NP + c],
                recv_sem=recv_sems.at[2 + NP + c],
                device_id=peer_x, device_id_type=pl.DeviceIdType.MESH)
            rdma_t.start()
            xtotal_rdmas.append(rdma_t)

        for c in range(NP):
            xtotal_rdmas[c].wait()
            rows = pl.ds(obase + c * PCHUNK, PCHUNK)
            out_ref[rows, :] = xrecv[c].astype(jnp.float32)

    return pl.pallas_call(
        body,
        out_shape=jax.ShapeDtypeStruct((T, D), jnp.float32),
        in_specs=[
            pl.BlockSpec(memory_space=pltpu.VMEM),
            pl.BlockSpec(memory_space=pltpu.VMEM),
            pl.BlockSpec(memory_space=pl.ANY),
            pl.BlockSpec(memory_space=pl.ANY),
        ],
        out_specs=pl.BlockSpec(memory_space=pltpu.VMEM),
        scratch_shapes=[
            pltpu.VMEM((HALF, D), jnp.bfloat16),
            pltpu.VMEM((HALF, 1), jnp.int32),
            pltpu.VMEM((HALF, D), jnp.bfloat16),
            pltpu.VMEM((HALF, 1), jnp.int32),
            pltpu.VMEM((NP, PCHUNK, D), jnp.bfloat16),
            pltpu.VMEM((NP, PCHUNK, D), jnp.bfloat16),
            pltpu.VMEM((NP, PCHUNK, D), jnp.bfloat16),
            pltpu.VMEM((NP, PCHUNK, D), jnp.bfloat16),
            pltpu.VMEM((2, D, FH), jnp.float32),
            pltpu.VMEM((2, FH, D), jnp.float32),
            pltpu.SemaphoreType.DMA((2 + 2 * NP,)),
            pltpu.SemaphoreType.DMA((2 + 2 * NP,)),
            pltpu.SemaphoreType.DMA((2,)),
            pltpu.SemaphoreType.DMA((2,)),
        ],
        compiler_params=pltpu.CompilerParams(
            collective_id=0, vmem_limit_bytes=48 * 1024 * 1024),
    )(x, a2d, W1, W2)
